# baseline (speedup 1.0000x reference)
import jax
import jax.numpy as jnp
from jax import lax
from jax.experimental import pallas as pl
from jax.experimental.pallas import tpu as pltpu
from jax.experimental.pallas import tpu_sc as plsc

BATCH = 16384
EMB_DIM = 128
NW = 32
B_PER_W = BATCH // NW
CHUNK = 128
N_CHUNKS = B_PER_W // CHUNK
VPR = EMB_DIM // 16
ROW_UNROLL = 2


def _body(idx_hbm, uid_table_hbm, iid_table_hbm, out_hbm,
          idx_v, u_bufs, v_bufs, out_buf, sem_u, sem_v):
    wid = lax.axis_index("s") * 2 + lax.axis_index("c")
    base = wid * 2 * N_CHUNKS

    pltpu.sync_copy(idx_hbm.at[pl.ds(base, 2 * N_CHUNKS)], idx_v)

    def start(j):
        s = lax.rem(j, 3)
        pltpu.make_async_copy(
            uid_table_hbm.at[idx_v.at[j]], u_bufs.at[s], sem_u.at[s]).start()
        pltpu.make_async_copy(
            iid_table_hbm.at[idx_v.at[N_CHUNKS + j]], v_bufs.at[s],
            sem_v.at[s]).start()

    def wait(s):
        pltpu.make_async_copy(
            uid_table_hbm.at[idx_v.at[0]], u_bufs.at[s], sem_u.at[s]).wait()
        pltpu.make_async_copy(
            iid_table_hbm.at[idx_v.at[0]], v_bufs.at[s], sem_v.at[s]).wait()

    lanes = lax.iota(jnp.int32, 16)
    last_lane = lanes == 15

    start(0)
    start(1)

    def chunk_body(j):
        s = lax.rem(j, 3)
        wait(s)

        @pl.when(j + 2 < N_CHUNKS)
        def _():
            start(j + 2)

        def group_body(g):
            r0 = g * ROW_UNROLL
            for i in range(ROW_UNROLL):
                row = r0 + i
                acc = u_bufs[s, row, pl.ds(0, 16)] * v_bufs[s, row, pl.ds(0, 16)]
                for k in range(1, VPR):
                    acc += (u_bufs[s, row, pl.ds(16 * k, 16)]
                            * v_bufs[s, row, pl.ds(16 * k, 16)])
                cum = plsc.cumsum(acc)
                plsc.store_compressed(out_buf.at[pl.ds(row, 16)], cum,
                                      mask=last_lane)

        pl.loop(0, CHUNK // ROW_UNROLL)(group_body)
        pltpu.sync_copy(out_buf.at[pl.ds(0, CHUNK)],
                        out_hbm.at[pl.ds(wid * B_PER_W + j * CHUNK, CHUNK)])

    pl.loop(0, N_CHUNKS)(chunk_body)


@jax.jit
def kernel(x, uid_table, iid_table):
    idx = (x.astype(jnp.int32)
           .reshape(NW, N_CHUNKS, CHUNK, 2)
           .transpose(0, 3, 1, 2)
           .reshape(NW * 2 * N_CHUNKS, CHUNK))

    mesh = plsc.VectorSubcoreMesh(core_axis_name="c", subcore_axis_name="s")
    run = pl.kernel(
        _body,
        out_type=jax.ShapeDtypeStruct((BATCH,), jnp.float32),
        mesh=mesh,
        compiler_params=pltpu.CompilerParams(needs_layout_passes=False),
        scratch_types=[
            pltpu.VMEM((2 * N_CHUNKS, CHUNK), jnp.int32),
            pltpu.VMEM((3, CHUNK, EMB_DIM), jnp.float32),
            pltpu.VMEM((3, CHUNK, EMB_DIM), jnp.float32),
            pltpu.VMEM((CHUNK + 16,), jnp.float32),
            pltpu.SemaphoreType.DMA((3,)),
            pltpu.SemaphoreType.DMA((3,)),
        ],
    )
    return run(idx, uid_table, iid_table)


# speedup vs baseline: 1.0432x; 1.0432x over previous
import jax
import jax.numpy as jnp
from jax import lax
from jax.experimental import pallas as pl
from jax.experimental.pallas import tpu as pltpu
from jax.experimental.pallas import tpu_sc as plsc

BATCH = 16384
EMB_DIM = 128
NW = 32
B_PER_W = BATCH // NW
CHUNK = 128
N_CHUNKS = B_PER_W // CHUNK
VPR = EMB_DIM // 16
ROW_UNROLL = 1


def _body(idx_hbm, uid_table_hbm, iid_table_hbm, out_hbm,
          idx_v, u_bufs, v_bufs, out_buf, sem_u, sem_v):
    wid = lax.axis_index("s") * 2 + lax.axis_index("c")
    base = wid * 2 * N_CHUNKS

    pltpu.sync_copy(idx_hbm.at[pl.ds(base, 2 * N_CHUNKS)], idx_v)

    def start(j, s):
        pltpu.make_async_copy(
            uid_table_hbm.at[idx_v.at[j]], u_bufs.at[s], sem_u).start()
        pltpu.make_async_copy(
            iid_table_hbm.at[idx_v.at[N_CHUNKS + j]], v_bufs.at[s], sem_v).start()

    def wait(s):
        pltpu.make_async_copy(
            uid_table_hbm.at[idx_v.at[0]], u_bufs.at[s], sem_u).wait()
        pltpu.make_async_copy(
            iid_table_hbm.at[idx_v.at[0]], v_bufs.at[s], sem_v).wait()

    lanes = lax.iota(jnp.int32, 16)
    last_lane = lanes == 15

    start(0, 0)

    def chunk_body(j):
        s = lax.rem(j, 2)
        wait(s)

        @pl.when(j + 1 < N_CHUNKS)
        def _():
            start(j + 1, 1 - s)

        def group_body(g):
            r0 = g * ROW_UNROLL
            for i in range(ROW_UNROLL):
                row = r0 + i
                acc = u_bufs[s, row, pl.ds(0, 16)] * v_bufs[s, row, pl.ds(0, 16)]
                for k in range(1, VPR):
                    acc += (u_bufs[s, row, pl.ds(16 * k, 16)]
                            * v_bufs[s, row, pl.ds(16 * k, 16)])
                cum = plsc.cumsum(acc)
                plsc.store_compressed(out_buf.at[pl.ds(row, 16)], cum,
                                      mask=last_lane)

        pl.loop(0, CHUNK // ROW_UNROLL)(group_body)
        pltpu.sync_copy(out_buf.at[pl.ds(0, CHUNK)],
                        out_hbm.at[pl.ds(wid * B_PER_W + j * CHUNK, CHUNK)])

    pl.loop(0, N_CHUNKS)(chunk_body)


@jax.jit
def kernel(x, uid_table, iid_table):
    idx = (x.astype(jnp.int32)
           .reshape(NW, N_CHUNKS, CHUNK, 2)
           .transpose(0, 3, 1, 2)
           .reshape(NW * 2 * N_CHUNKS, CHUNK))

    mesh = plsc.VectorSubcoreMesh(core_axis_name="c", subcore_axis_name="s")
    run = pl.kernel(
        _body,
        out_type=jax.ShapeDtypeStruct((BATCH,), jnp.float32),
        mesh=mesh,
        compiler_params=pltpu.CompilerParams(needs_layout_passes=False),
        scratch_types=[
            pltpu.VMEM((2 * N_CHUNKS, CHUNK), jnp.int32),
            pltpu.VMEM((2, CHUNK, EMB_DIM), jnp.float32),
            pltpu.VMEM((2, CHUNK, EMB_DIM), jnp.float32),
            pltpu.VMEM((CHUNK + 16,), jnp.float32),
            pltpu.SemaphoreType.DMA,
            pltpu.SemaphoreType.DMA,
        ],
    )
    return run(idx, uid_table, iid_table)


# speedup vs baseline: 1.0558x; 1.0120x over previous
import jax
import jax.numpy as jnp
from jax import lax
from jax.experimental import pallas as pl
from jax.experimental.pallas import tpu as pltpu
from jax.experimental.pallas import tpu_sc as plsc

BATCH = 16384
EMB_DIM = 128
NW = 32
B_PER_W = BATCH // NW
CHUNK = 128
N_CHUNKS = B_PER_W // CHUNK
VPR = EMB_DIM // 16
ROW_UNROLL = 1


def _body(idx_hbm, uid_table_hbm, iid_table_hbm, out_hbm,
          idx_v, u_bufs, v_bufs, out_buf, sem_u, sem_v):
    wid = lax.axis_index("s") * 2 + lax.axis_index("c")
    base = wid * 2 * N_CHUNKS

    pltpu.sync_copy(idx_hbm.at[pl.ds(base, 2 * N_CHUNKS)], idx_v)

    def start(j, s):
        pltpu.make_async_copy(
            uid_table_hbm.at[idx_v.at[j]], u_bufs.at[s], sem_u).start()
        pltpu.make_async_copy(
            iid_table_hbm.at[idx_v.at[N_CHUNKS + j]], v_bufs.at[s], sem_v).start()

    def wait(s):
        pltpu.make_async_copy(
            uid_table_hbm.at[idx_v.at[0]], u_bufs.at[s], sem_u).wait()
        pltpu.make_async_copy(
            iid_table_hbm.at[idx_v.at[0]], v_bufs.at[s], sem_v).wait()

    lanes = lax.iota(jnp.int32, 16)
    last_lane = lanes == 15

    start(0, 0)

    def chunk_body(j):
        s = lax.rem(j, 2)
        wait(s)

        @pl.when(j + 1 < N_CHUNKS)
        def _():
            start(j + 1, 1 - s)

        obase = j * CHUNK

        def group_body(g):
            r0 = g * ROW_UNROLL
            for i in range(ROW_UNROLL):
                row = r0 + i
                acc = u_bufs[s, row, pl.ds(0, 16)] * v_bufs[s, row, pl.ds(0, 16)]
                for k in range(1, VPR):
                    acc += (u_bufs[s, row, pl.ds(16 * k, 16)]
                            * v_bufs[s, row, pl.ds(16 * k, 16)])
                cum = plsc.cumsum(acc)
                plsc.store_compressed(out_buf.at[pl.ds(obase + row, 16)], cum,
                                      mask=last_lane)

        pl.loop(0, CHUNK // ROW_UNROLL)(group_body)

    pl.loop(0, N_CHUNKS)(chunk_body)
    pltpu.sync_copy(out_buf.at[pl.ds(0, B_PER_W)],
                    out_hbm.at[pl.ds(wid * B_PER_W, B_PER_W)])


@jax.jit
def kernel(x, uid_table, iid_table):
    idx = (x.astype(jnp.int32)
           .reshape(NW, N_CHUNKS, CHUNK, 2)
           .transpose(0, 3, 1, 2)
           .reshape(NW * 2 * N_CHUNKS, CHUNK))

    mesh = plsc.VectorSubcoreMesh(core_axis_name="c", subcore_axis_name="s")
    run = pl.kernel(
        _body,
        out_type=jax.ShapeDtypeStruct((BATCH,), jnp.float32),
        mesh=mesh,
        compiler_params=pltpu.CompilerParams(needs_layout_passes=False),
        scratch_types=[
            pltpu.VMEM((2 * N_CHUNKS, CHUNK), jnp.int32),
            pltpu.VMEM((2, CHUNK, EMB_DIM), jnp.float32),
            pltpu.VMEM((2, CHUNK, EMB_DIM), jnp.float32),
            pltpu.VMEM((B_PER_W + 16,), jnp.float32),
            pltpu.SemaphoreType.DMA,
            pltpu.SemaphoreType.DMA,
        ],
    )
    return run(idx, uid_table, iid_table)
